# baseline (speedup 1.0000x reference)
import functools

import jax
import jax.numpy as jnp
from jax import lax
from jax.experimental import pallas as pl
from jax.experimental.pallas import tpu as pltpu
from jax.experimental.pallas import tpu_sc as plsc

N = 10000
E = 320000
H = 64
HP = 128

NC = 2
NS = 16
NW = NC * NS
EPW = E // NW
CHUNK = 80
NCHUNK = EPW // CHUNK
N2 = N // 2
N2_PAD = 5120
ROWS_PER_TILE = N2_PAD // NS
ZROWS = 64


def _sc_edge_aggregate(xproj2, ceproj2, senders, receivers):
  mesh = plsc.VectorSubcoreMesh(core_axis_name="c", subcore_axis_name="s")

  @functools.partial(
      pl.kernel,
      out_type=jax.ShapeDtypeStruct((NC, N2_PAD, HP), jnp.float32),
      mesh=mesh,
      scratch_types=[
          pltpu.VMEM((1, CHUNK), jnp.int32),
          pltpu.VMEM((1, CHUNK), jnp.int32),
          pltpu.VMEM((1, CHUNK), jnp.int32),
          pltpu.VMEM((1, CHUNK), jnp.int32),
          pltpu.VMEM((1, CHUNK), jnp.int32),
          pltpu.VMEM((1, CHUNK), jnp.int32),
          pltpu.VMEM((1, CHUNK), jnp.int32),
          pltpu.VMEM((1, CHUNK), jnp.int32),
          pltpu.VMEM((CHUNK // 2, HP), jnp.float32),
          pltpu.VMEM((CHUNK // 2, HP), jnp.float32),
          pltpu.VMEM((CHUNK, HP), jnp.float32),
          pltpu.VMEM((CHUNK, HP), jnp.float32),
          pltpu.VMEM((CHUNK, HP), jnp.float32),
          pltpu.VMEM((ZROWS, HP), jnp.float32),
          pltpu.VMEM_SHARED((N2_PAD, HP), jnp.float32),
          pltpu.SemaphoreType.DMA,
          pltpu.SemaphoreType.DMA,
          pltpu.SemaphoreType.DMA,
          pltpu.SemaphoreType.DMA,
      ],
  )
  def k(xproj_hbm, ceproj_hbm, snd_hbm, rcv_hbm, out_hbm,
        sidx0, sidx1, ridx0, ridx1, sh0, sh1, rh0, rh1,
        ce0, ce1, xs0, xs1, he, zbuf, acc,
        sem_ce0, sem_ce1, sem_g0, sem_g1):
    c = lax.axis_index("c")
    s = lax.axis_index("s")
    wid = s * NC + c
    ebase = wid * EPW
    pbase = wid * (EPW // 2)
    sidx = (sidx0, sidx1)
    ridx = (ridx0, ridx1)
    sh = (sh0, sh1)
    rh = (rh0, rh1)
    ce = (ce0, ce1)
    xs = (xs0, xs1)
    sem_ce = (sem_ce0, sem_ce1)
    sem_g = (sem_g0, sem_g1)

    def zrow(r, carry):
      for q in range(HP // 16):
        zbuf[r, pl.ds(q * 16, 16)] = jnp.zeros((16,), jnp.float32)
      return carry
    lax.fori_loop(0, ZROWS, zrow, 0)
    for kk in range(ROWS_PER_TILE // ZROWS):
      pltpu.sync_copy(zbuf, acc.at[pl.ds(s * ROWS_PER_TILE + kk * ZROWS, ZROWS)])
    plsc.subcore_barrier()

    def issue(j, p):
      pltpu.sync_copy(snd_hbm.at[pl.ds(ebase + j * CHUNK, CHUNK)], sidx[p].at[0])
      pltpu.sync_copy(rcv_hbm.at[pl.ds(ebase + j * CHUNK, CHUNK)], ridx[p].at[0])
      for q in range(CHUNK // 16):
        sl = pl.ds(q * 16, 16)
        sh[p][0, sl] = lax.shift_right_logical(sidx[p][0, sl], 1)
        rh[p][0, sl] = lax.shift_right_logical(ridx[p][0, sl], 1)
      pltpu.async_copy(ceproj_hbm.at[pl.ds(pbase + j * (CHUNK // 2), CHUNK // 2)],
                       ce[p], sem_ce[p])
      pltpu.async_copy(xproj_hbm.at[sh[p].at[0]], xs[p], sem_g[p])

    def process(j, p):
      pltpu.make_async_copy(
          ceproj_hbm.at[pl.ds(pbase + j * (CHUNK // 2), CHUNK // 2)],
          ce[p], sem_ce[p]).wait()
      pltpu.make_async_copy(xproj_hbm.at[pl.ds(0, CHUNK)],
                            xs[p], sem_g[p]).wait()

      def group(g, rc):
        sv = sidx[p][0, pl.ds(g * 16, 16)]
        rv = ridx[p][0, pl.ds(g * 16, 16)]
        base_r = g * 16
        base_rp = g * 8
        for lane in range(16):
          r = base_r + lane
          rp = base_rp + lane // 2
          soff = (sv[lane] & 1) * 64
          roff = (rv[lane] & 1) * 64
          zoff = 64 - roff
          for q in range(H // 16):
            cv = ce[p][rp, pl.ds((lane % 2) * 64 + q * 16, 16)]
            xv = xs[p][r, pl.ds(soff + q * 16, 16)]
            he[r, pl.ds(roff + q * 16, 16)] = jnp.maximum(cv + xv, 0.0)
            he[r, pl.ds(zoff + q * 16, 16)] = jnp.zeros((16,), jnp.float32)
        return rc
      lax.fori_loop(0, CHUNK // 16, group, 0)

      pltpu.sync_copy(he, acc.at[rh[p].at[0]], add=True)

    issue(0, 0)

    def two(t, carry):
      j = 2 * t
      issue(j + 1, 1)
      process(j, 0)
      issue(j + 2, 0)
      process(j + 1, 1)
      return carry
    lax.fori_loop(0, (NCHUNK - 1) // 2, two, 0)
    process(NCHUNK - 1, 0)
    plsc.subcore_barrier()

    pltpu.sync_copy(acc.at[pl.ds(s * ROWS_PER_TILE, ROWS_PER_TILE)],
                    out_hbm.at[c, pl.ds(s * ROWS_PER_TILE, ROWS_PER_TILE)])

  return k(xproj2, ceproj2, senders, receivers)


def _proj_body(x2_ref, w2x_ref, g_ref, wg_ref, b_ref, e2_ref, w2e_ref,
               xo_ref, ceo_ref):
  i = pl.program_id(0)

  @pl.when(i == 0)
  def _():
    cst = jnp.dot(g_ref[...], wg_ref[...], preferred_element_type=jnp.float32)
    cst2 = jnp.concatenate([cst + b_ref[...], cst + b_ref[...]], axis=1)
    xo_ref[...] = (jnp.dot(x2_ref[...], w2x_ref[...],
                           preferred_element_type=jnp.float32) + cst2)

  @pl.when(i > 0)
  def _():
    ceo_ref[...] = jnp.dot(e2_ref[...], w2e_ref[...],
                           preferred_element_type=jnp.float32)


def _node_body(aggp_ref, x2_ref, wa_ref, wx_ref, g_ref, wg_ref, b_ref,
               wg1_ref, wg2_ref, wg3_ref, bgb_ref,
               wa2_ref, wv2_ref, wgn_ref, b2a_ref, w2b_ref, b2b_ref,
               out_ref, gnew_ref, agg_s, hv_s, acc_ref):
  i = pl.program_id(0)
  H_ = 64
  IN_X = 160
  BP = aggp_ref.shape[1]

  @pl.when(i < 5)
  def _():
    pk = aggp_ref[0] + aggp_ref[1]
    agg_s[pl.ds(i * BP, BP), :] = pk
    cst = jnp.dot(g_ref[...], wg_ref[...], preferred_element_type=jnp.float32)
    hv_e = jnp.maximum(
        jnp.dot(pk[:, :H_], wa_ref[...], preferred_element_type=jnp.float32)
        + jnp.dot(x2_ref[:, :IN_X], wx_ref[...],
                  preferred_element_type=jnp.float32)
        + cst + b_ref[...], 0.0)
    hv_o = jnp.maximum(
        jnp.dot(pk[:, H_:], wa_ref[...], preferred_element_type=jnp.float32)
        + jnp.dot(x2_ref[:, IN_X:], wx_ref[...],
                  preferred_element_type=jnp.float32)
        + cst + b_ref[...], 0.0)
    hv_s[pl.ds(i * BP, BP), :] = jnp.concatenate([hv_e, hv_o], axis=1)
    part = jnp.concatenate(
        [jnp.sum(pk[:, :H_], axis=0, keepdims=True)
         + jnp.sum(pk[:, H_:], axis=0, keepdims=True),
         jnp.sum(hv_e, axis=0, keepdims=True)
         + jnp.sum(hv_o, axis=0, keepdims=True)], axis=0)

    @pl.when(i == 0)
    def _():
      acc_ref[...] = jnp.zeros_like(acc_ref)

    acc_ref[0:2, 0:H_] += part

  @pl.when(i >= 5)
  def _():
    ii = i - 5
    pk = agg_s[pl.ds(ii * BP, BP), :]
    hvpk = hv_s[pl.ds(ii * BP, BP), :]
    mean_he = acc_ref[0:1, 0:H_] * (1.0 / E)
    mean_hv = acc_ref[1:2, 0:H_] * (1.0 / N)
    g_new = jnp.maximum(
        jnp.dot(mean_he, wg1_ref[...], preferred_element_type=jnp.float32)
        + jnp.dot(mean_hv, wg2_ref[...], preferred_element_type=jnp.float32)
        + jnp.dot(g_ref[...], wg3_ref[...], preferred_element_type=jnp.float32)
        + bgb_ref[...], 0.0)
    gterm = (jnp.dot(g_new, wgn_ref[...], preferred_element_type=jnp.float32)
             + b2a_ref[...])
    h2_e = jnp.maximum(
        jnp.dot(pk[:, :H_], wa2_ref[...], preferred_element_type=jnp.float32)
        + jnp.dot(hvpk[:, :H_], wv2_ref[...],
                  preferred_element_type=jnp.float32) + gterm, 0.0)
    h2_o = jnp.maximum(
        jnp.dot(pk[:, H_:], wa2_ref[...], preferred_element_type=jnp.float32)
        + jnp.dot(hvpk[:, H_:], wv2_ref[...],
                  preferred_element_type=jnp.float32) + gterm, 0.0)
    out_e = (jnp.dot(h2_e, w2b_ref[...], preferred_element_type=jnp.float32)
             + b2b_ref[...])
    out_o = (jnp.dot(h2_o, w2b_ref[...], preferred_element_type=jnp.float32)
             + b2b_ref[...])
    pair = jnp.concatenate([out_e, out_o], axis=1)
    out_ref[...] = pair.reshape(2 * BP, out_e.shape[1])

    @pl.when(i == 5)
    def _():
      gnew_ref[...] = g_new


def _full(shape):
  nd = len(shape)
  return pl.BlockSpec(shape, lambda i: (0,) * nd)


def _blockdiag2(w):
  z = jnp.zeros_like(w)
  return jnp.concatenate([jnp.concatenate([w, z], axis=1),
                          jnp.concatenate([z, w], axis=1)], axis=0)


def kernel(cat_x, cat_e, edge_index, global_attr, W_eb, b_eb, W_nb, b_nb,
           W_gb, b_gb, W_n2a, b_n2a, W_n2b, b_n2b):
  IN_X = cat_x.shape[1]
  IN_E = cat_e.shape[1]
  G = global_attr.shape[0]
  senders = edge_index[0]
  receivers = edge_index[1]
  g_row = global_attr.reshape(1, G)
  cat_x2 = cat_x.reshape(N2, 2 * IN_X)

  W_eb_e = W_eb[:IN_E]
  W_eb_x = W_eb[IN_E:IN_E + IN_X]
  W_eb_g = W_eb[IN_E + IN_X:]

  BE2 = 4000
  E2 = E // 2
  xproj2, ceproj2 = pl.pallas_call(
      _proj_body,
      grid=(1 + E2 // BE2,),
      in_specs=[_full((N2, 2 * IN_X)), _full((2 * IN_X, HP)),
                _full((1, G)), _full((G, H)), _full((1, H)),
                pl.BlockSpec((BE2, 2 * IN_E),
                             lambda i: (jnp.maximum(i - 1, 0), 0)),
                _full((2 * IN_E, HP))],
      out_specs=[_full((N2, HP)),
                 pl.BlockSpec((BE2, HP), lambda i: (jnp.maximum(i - 1, 0), 0))],
      out_shape=[jax.ShapeDtypeStruct((N2, HP), jnp.float32),
                 jax.ShapeDtypeStruct((E2, HP), jnp.float32)],
  )(cat_x2, _blockdiag2(W_eb_x), g_row, W_eb_g, b_eb.reshape(1, H),
    cat_e.reshape(E2, 2 * IN_E), _blockdiag2(W_eb_e))

  agg_packed = _sc_edge_aggregate(xproj2, ceproj2, senders, receivers)

  W_nb_a = W_nb[:H]
  W_nb_x = W_nb[H:H + IN_X]
  W_nb_g = W_nb[H + IN_X:]
  OUT = W_n2b.shape[1]
  W_gb1 = W_gb[:H]
  W_gb2 = W_gb[H:2 * H]
  W_gb3 = W_gb[2 * H:]
  W_n2a_a = W_n2a[:H]
  W_n2a_v = W_n2a[H:2 * H]
  W_n2a_g = W_n2a[2 * H:]
  BP = 1000
  BN = 2 * BP
  out_nodes, g_new = pl.pallas_call(
      _node_body,
      grid=(10,),
      in_specs=[pl.BlockSpec((NC, BP, HP), lambda i: (0, jnp.minimum(i, 4), 0)),
                pl.BlockSpec((BP, 2 * IN_X), lambda i: (jnp.minimum(i, 4), 0)),
                _full((H, H)), _full((IN_X, H)), _full((1, G)), _full((G, H)),
                _full((1, H)),
                _full((H, G)), _full((H, G)), _full((G, G)), _full((1, G)),
                _full((H, H)), _full((H, H)), _full((G, H)), _full((1, H)),
                _full((H, OUT)), _full((1, OUT))],
      out_specs=[pl.BlockSpec((BN, OUT), lambda i: (jnp.maximum(i - 5, 0), 0)),
                 _full((1, G))],
      out_shape=[jax.ShapeDtypeStruct((N, OUT), jnp.float32),
                 jax.ShapeDtypeStruct((1, G), jnp.float32)],
      scratch_shapes=[pltpu.VMEM((N2, HP), jnp.float32),
                      pltpu.VMEM((N2, HP), jnp.float32),
                      pltpu.VMEM((8, 128), jnp.float32)],
  )(agg_packed[:, :N2, :], cat_x2, W_nb_a, W_nb_x, g_row, W_nb_g,
    b_nb.reshape(1, H),
    W_gb1, W_gb2, W_gb3, b_gb.reshape(1, G),
    W_n2a_a, W_n2a_v, W_n2a_g, b_n2a.reshape(1, H),
    W_n2b, b_n2b.reshape(1, OUT))

  return (out_nodes, g_new.reshape(G))


# speedup vs baseline: 1.1477x; 1.1477x over previous
import functools

import jax
import jax.numpy as jnp
from jax import lax
from jax.experimental import pallas as pl
from jax.experimental.pallas import tpu as pltpu
from jax.experimental.pallas import tpu_sc as plsc

N = 10000
E = 320000
H = 64
HP = 128

NC = 2
NS = 16
NW = NC * NS
EPW = E // NW
CHUNK = 80
NCHUNK = EPW // CHUNK
N_PAD = 10240
ROWS_PER_TILE = N_PAD // NS
ZROWS = 128


def _sc_edge_aggregate(xproj, ceproj, senders, receivers):
  mesh = plsc.VectorSubcoreMesh(core_axis_name="c", subcore_axis_name="s")

  @functools.partial(
      pl.kernel,
      out_type=jax.ShapeDtypeStruct((NC, N_PAD, HP), jnp.float32),
      mesh=mesh,
      scratch_types=[
          pltpu.VMEM((1, CHUNK), jnp.int32),
          pltpu.VMEM((1, CHUNK), jnp.int32),
          pltpu.VMEM((CHUNK, H), jnp.float32),
          pltpu.VMEM((CHUNK, HP), jnp.float32),
          pltpu.VMEM((ZROWS, HP), jnp.float32),
          pltpu.VMEM_SHARED((N_PAD, HP), jnp.float32),
          pltpu.SemaphoreType.DMA,
          pltpu.SemaphoreType.DMA,
      ],
  )
  def k(xproj_hbm, ceproj_hbm, snd_hbm, rcv_hbm, out_hbm,
        sidx, ridx, ce, xs, zbuf, acc, sem_ce, sem_g):
    c = lax.axis_index("c")
    s = lax.axis_index("s")
    wid = s * NC + c
    ebase = wid * EPW

    def zrow(r, carry):
      for q in range(HP // 16):
        zbuf[r, pl.ds(q * 16, 16)] = jnp.zeros((16,), jnp.float32)
      return carry
    lax.fori_loop(0, ZROWS, zrow, 0)
    for kk in range(ROWS_PER_TILE // ZROWS):
      pltpu.sync_copy(zbuf, acc.at[pl.ds(s * ROWS_PER_TILE + kk * ZROWS, ZROWS)])
    plsc.subcore_barrier()

    def step(j, carry):
      pltpu.sync_copy(snd_hbm.at[pl.ds(ebase + j * CHUNK, CHUNK)], sidx.at[0])
      pltpu.sync_copy(rcv_hbm.at[pl.ds(ebase + j * CHUNK, CHUNK)], ridx.at[0])
      pltpu.async_copy(ceproj_hbm.at[pl.ds(ebase + j * CHUNK, CHUNK)],
                       ce, sem_ce)
      pltpu.async_copy(xproj_hbm.at[sidx.at[0]], xs, sem_g)
      pltpu.make_async_copy(ceproj_hbm.at[pl.ds(ebase + j * CHUNK, CHUNK)],
                            ce, sem_ce).wait()
      pltpu.make_async_copy(xproj_hbm.at[pl.ds(0, CHUNK)], xs, sem_g).wait()

      def row(r, rc):
        for q in range(H // 16):
          sl = pl.ds(q * 16, 16)
          xs[r, sl] = jnp.maximum(ce[r, sl] + xs[r, sl], 0.0)
        return rc
      lax.fori_loop(0, CHUNK, row, 0)

      pltpu.sync_copy(xs, acc.at[ridx.at[0]], add=True)
      return carry
    lax.fori_loop(0, NCHUNK, step, 0)
    plsc.subcore_barrier()

    pltpu.sync_copy(acc.at[pl.ds(s * ROWS_PER_TILE, ROWS_PER_TILE)],
                    out_hbm.at[c, pl.ds(s * ROWS_PER_TILE, ROWS_PER_TILE)])

  return k(xproj, ceproj, senders, receivers)


def _proj_body(x_ref, wx_ref, g_ref, wg_ref, b_ref, e_ref, we_ref,
               xo_ref, ceo_ref):
  i = pl.program_id(0)

  @pl.when(i == 0)
  def _():
    cst = jnp.dot(g_ref[...], wg_ref[...], preferred_element_type=jnp.float32)
    proj = (jnp.dot(x_ref[...], wx_ref[...],
                    preferred_element_type=jnp.float32) + cst + b_ref[...])
    xo_ref[...] = jnp.concatenate(
        [proj, jnp.zeros((proj.shape[0], HP - H), jnp.float32)], axis=1)

  @pl.when(i > 0)
  def _():
    ceo_ref[...] = jnp.dot(e_ref[...], we_ref[...],
                           preferred_element_type=jnp.float32)


def _node_body(aggp_ref, x_ref, wa_ref, wx_ref, g_ref, wg_ref, b_ref,
               wg1_ref, wg2_ref, wg3_ref, bgb_ref,
               wa2_ref, wv2_ref, wgn_ref, b2a_ref, w2b_ref, b2b_ref,
               out_ref, gnew_ref, agg_s, hv_s, acc_ref):
  i = pl.program_id(0)
  BN = aggp_ref.shape[1]

  @pl.when(i < 5)
  def _():
    pk = aggp_ref[0, :, :H] + aggp_ref[1, :, :H]
    agg_s[pl.ds(i * BN, BN), :] = pk
    cst = jnp.dot(g_ref[...], wg_ref[...], preferred_element_type=jnp.float32)
    hv = jnp.maximum(
        jnp.dot(pk, wa_ref[...], preferred_element_type=jnp.float32)
        + jnp.dot(x_ref[...], wx_ref[...], preferred_element_type=jnp.float32)
        + cst + b_ref[...], 0.0)
    hv_s[pl.ds(i * BN, BN), :] = hv
    part = jnp.concatenate(
        [jnp.sum(pk, axis=0, keepdims=True),
         jnp.sum(hv, axis=0, keepdims=True)], axis=0)

    @pl.when(i == 0)
    def _():
      acc_ref[...] = jnp.zeros_like(acc_ref)

    acc_ref[0:2, 0:H] += part

  @pl.when(i >= 5)
  def _():
    ii = i - 5
    pk = agg_s[pl.ds(ii * BN, BN), :]
    hv = hv_s[pl.ds(ii * BN, BN), :]
    mean_he = acc_ref[0:1, 0:H] * (1.0 / E)
    mean_hv = acc_ref[1:2, 0:H] * (1.0 / N)
    g_new = jnp.maximum(
        jnp.dot(mean_he, wg1_ref[...], preferred_element_type=jnp.float32)
        + jnp.dot(mean_hv, wg2_ref[...], preferred_element_type=jnp.float32)
        + jnp.dot(g_ref[...], wg3_ref[...], preferred_element_type=jnp.float32)
        + bgb_ref[...], 0.0)
    gterm = (jnp.dot(g_new, wgn_ref[...], preferred_element_type=jnp.float32)
             + b2a_ref[...])
    h2 = jnp.maximum(
        jnp.dot(pk, wa2_ref[...], preferred_element_type=jnp.float32)
        + jnp.dot(hv, wv2_ref[...], preferred_element_type=jnp.float32)
        + gterm, 0.0)
    out_ref[...] = (jnp.dot(h2, w2b_ref[...], preferred_element_type=jnp.float32)
                    + b2b_ref[...])

    @pl.when(i == 5)
    def _():
      gnew_ref[...] = g_new


def _full(shape):
  nd = len(shape)
  return pl.BlockSpec(shape, lambda i: (0,) * nd)


def kernel(cat_x, cat_e, edge_index, global_attr, W_eb, b_eb, W_nb, b_nb,
           W_gb, b_gb, W_n2a, b_n2a, W_n2b, b_n2b):
  IN_X = cat_x.shape[1]
  IN_E = cat_e.shape[1]
  G = global_attr.shape[0]
  senders = edge_index[0]
  receivers = edge_index[1]
  g_row = global_attr.reshape(1, G)

  W_eb_e = W_eb[:IN_E]
  W_eb_x = W_eb[IN_E:IN_E + IN_X]
  W_eb_g = W_eb[IN_E + IN_X:]

  BE = 8000
  xproj, ceproj = pl.pallas_call(
      _proj_body,
      grid=(1 + E // BE,),
      in_specs=[_full((N, IN_X)), _full((IN_X, H)),
                _full((1, G)), _full((G, H)), _full((1, H)),
                pl.BlockSpec((BE, IN_E), lambda i: (jnp.maximum(i - 1, 0), 0)),
                _full((IN_E, H))],
      out_specs=[_full((N, HP)),
                 pl.BlockSpec((BE, H), lambda i: (jnp.maximum(i - 1, 0), 0))],
      out_shape=[jax.ShapeDtypeStruct((N, HP), jnp.float32),
                 jax.ShapeDtypeStruct((E, H), jnp.float32)],
  )(cat_x, W_eb_x, g_row, W_eb_g, b_eb.reshape(1, H), cat_e, W_eb_e)

  agg_packed = _sc_edge_aggregate(xproj, ceproj, senders, receivers)

  W_nb_a = W_nb[:H]
  W_nb_x = W_nb[H:H + IN_X]
  W_nb_g = W_nb[H + IN_X:]
  OUT = W_n2b.shape[1]
  W_gb1 = W_gb[:H]
  W_gb2 = W_gb[H:2 * H]
  W_gb3 = W_gb[2 * H:]
  W_n2a_a = W_n2a[:H]
  W_n2a_v = W_n2a[H:2 * H]
  W_n2a_g = W_n2a[2 * H:]
  BN = 2000
  out_nodes, g_new = pl.pallas_call(
      _node_body,
      grid=(10,),
      in_specs=[pl.BlockSpec((NC, BN, HP), lambda i: (0, jnp.minimum(i, 4), 0)),
                pl.BlockSpec((BN, IN_X), lambda i: (jnp.minimum(i, 4), 0)),
                _full((H, H)), _full((IN_X, H)), _full((1, G)), _full((G, H)),
                _full((1, H)),
                _full((H, G)), _full((H, G)), _full((G, G)), _full((1, G)),
                _full((H, H)), _full((H, H)), _full((G, H)), _full((1, H)),
                _full((H, OUT)), _full((1, OUT))],
      out_specs=[pl.BlockSpec((BN, OUT), lambda i: (jnp.maximum(i - 5, 0), 0)),
                 _full((1, G))],
      out_shape=[jax.ShapeDtypeStruct((N, OUT), jnp.float32),
                 jax.ShapeDtypeStruct((1, G), jnp.float32)],
      scratch_shapes=[pltpu.VMEM((N, H), jnp.float32),
                      pltpu.VMEM((N, H), jnp.float32),
                      pltpu.VMEM((8, 128), jnp.float32)],
  )(agg_packed, cat_x, W_nb_a, W_nb_x, g_row, W_nb_g, b_nb.reshape(1, H),
    W_gb1, W_gb2, W_gb3, b_gb.reshape(1, G),
    W_n2a_a, W_n2a_v, W_n2a_g, b_n2a.reshape(1, H),
    W_n2b, b_n2b.reshape(1, OUT))

  return (out_nodes, g_new.reshape(G))


# speedup vs baseline: 1.3705x; 1.1941x over previous
import functools

import jax
import jax.numpy as jnp
from jax import lax
from jax.experimental import pallas as pl
from jax.experimental.pallas import tpu as pltpu
from jax.experimental.pallas import tpu_sc as plsc

N = 10000
E = 320000
H = 64
HP = 128

NC = 2
NS = 16
NW = NC * NS
EPW = E // NW
CHUNK = 80
NCHUNK = EPW // CHUNK
N_PAD = 10240
ROWS_PER_TILE = N_PAD // NS
ZROWS = 128


def _sc_edge_aggregate(xproj, ceproj, senders, receivers):
  mesh = plsc.VectorSubcoreMesh(core_axis_name="c", subcore_axis_name="s")

  @functools.partial(
      pl.kernel,
      out_type=jax.ShapeDtypeStruct((NC, N_PAD, HP), jnp.float32),
      mesh=mesh,
      scratch_types=[
          pltpu.VMEM((1, CHUNK), jnp.int32),
          pltpu.VMEM((1, CHUNK), jnp.int32),
          pltpu.VMEM((1, CHUNK), jnp.int32),
          pltpu.VMEM((1, CHUNK), jnp.int32),
          pltpu.VMEM((CHUNK, H), jnp.float32),
          pltpu.VMEM((CHUNK, HP), jnp.float32),
          pltpu.VMEM((ZROWS, HP), jnp.float32),
          pltpu.VMEM_SHARED((N_PAD, HP), jnp.float32),
          pltpu.SemaphoreType.DMA,
          pltpu.SemaphoreType.DMA,
          pltpu.SemaphoreType.DMA,
          pltpu.SemaphoreType.DMA,
          pltpu.SemaphoreType.DMA,
          pltpu.SemaphoreType.DMA,
      ],
  )
  def k(xproj_hbm, ceproj_hbm, snd_hbm, rcv_hbm, out_hbm,
        sidx0, sidx1, ridx0, ridx1, ce, xs, zbuf, acc,
        sem_ce, sem_g, sem_s0, sem_s1, sem_r0, sem_r1):
    c = lax.axis_index("c")
    s = lax.axis_index("s")
    wid = s * NC + c
    ebase = wid * EPW
    sidx = (sidx0, sidx1)
    ridx = (ridx0, ridx1)
    sem_s = (sem_s0, sem_s1)
    sem_r = (sem_r0, sem_r1)

    def zrow(r, carry):
      for q in range(HP // 16):
        zbuf[r, pl.ds(q * 16, 16)] = jnp.zeros((16,), jnp.float32)
      return carry
    lax.fori_loop(0, ZROWS, zrow, 0)
    for kk in range(ROWS_PER_TILE // ZROWS):
      pltpu.sync_copy(zbuf, acc.at[pl.ds(s * ROWS_PER_TILE + kk * ZROWS, ZROWS)])
    plsc.subcore_barrier()

    def fetch_idx(j, p):
      pltpu.async_copy(snd_hbm.at[pl.ds(ebase + j * CHUNK, CHUNK)],
                       sidx[p].at[0], sem_s[p])
      pltpu.async_copy(rcv_hbm.at[pl.ds(ebase + j * CHUNK, CHUNK)],
                       ridx[p].at[0], sem_r[p])

    def work(j, p):
      pltpu.make_async_copy(snd_hbm.at[pl.ds(0, CHUNK)],
                            sidx[p].at[0], sem_s[p]).wait()
      pltpu.async_copy(ceproj_hbm.at[pl.ds(ebase + j * CHUNK, CHUNK)],
                       ce, sem_ce)
      pltpu.async_copy(xproj_hbm.at[sidx[p].at[0]], xs, sem_g)
      pltpu.make_async_copy(ceproj_hbm.at[pl.ds(ebase + j * CHUNK, CHUNK)],
                            ce, sem_ce).wait()
      pltpu.make_async_copy(xproj_hbm.at[pl.ds(0, CHUNK)], xs, sem_g).wait()

      def row(r, rc):
        for q in range(H // 16):
          sl = pl.ds(q * 16, 16)
          xs[r, sl] = jnp.maximum(ce[r, sl] + xs[r, sl], 0.0)
        return rc
      lax.fori_loop(0, CHUNK, row, 0)

      pltpu.make_async_copy(rcv_hbm.at[pl.ds(0, CHUNK)],
                            ridx[p].at[0], sem_r[p]).wait()
      pltpu.sync_copy(xs, acc.at[ridx[p].at[0]], add=True)

    fetch_idx(0, 0)

    def two(t, carry):
      j = 2 * t
      fetch_idx(j + 1, 1)
      work(j, 0)
      fetch_idx(j + 2, 0)
      work(j + 1, 1)
      return carry
    lax.fori_loop(0, (NCHUNK - 1) // 2, two, 0)
    work(NCHUNK - 1, 0)
    plsc.subcore_barrier()

    pltpu.sync_copy(acc.at[pl.ds(s * ROWS_PER_TILE, ROWS_PER_TILE)],
                    out_hbm.at[c, pl.ds(s * ROWS_PER_TILE, ROWS_PER_TILE)])

  return k(xproj, ceproj, senders, receivers)


def _proj_body(x_ref, wx_ref, g_ref, wg_ref, b_ref, e_ref, we_ref,
               xo_ref, ceo_ref):
  i = pl.program_id(0)

  @pl.when(i == 0)
  def _():
    cst = jnp.dot(g_ref[...], wg_ref[...], preferred_element_type=jnp.float32)
    proj = (jnp.dot(x_ref[...], wx_ref[...],
                    preferred_element_type=jnp.float32) + cst + b_ref[...])
    xo_ref[...] = jnp.concatenate(
        [proj, jnp.zeros((proj.shape[0], HP - H), jnp.float32)], axis=1)

  @pl.when(i > 0)
  def _():
    ceo_ref[...] = jnp.dot(e_ref[...], we_ref[...],
                           preferred_element_type=jnp.float32)


def _node_body(aggp_ref, x_ref, wa_ref, wx_ref, g_ref, wg_ref, b_ref,
               wg1_ref, wg2_ref, wg3_ref, bgb_ref,
               wa2_ref, wv2_ref, wgn_ref, b2a_ref, w2b_ref, b2b_ref,
               out_ref, gnew_ref, agg_s, hv_s, acc_ref):
  i = pl.program_id(0)
  BN = aggp_ref.shape[1]

  @pl.when(i < 5)
  def _():
    pk = aggp_ref[0, :, :H] + aggp_ref[1, :, :H]
    agg_s[pl.ds(i * BN, BN), :] = pk
    cst = jnp.dot(g_ref[...], wg_ref[...], preferred_element_type=jnp.float32)
    hv = jnp.maximum(
        jnp.dot(pk, wa_ref[...], preferred_element_type=jnp.float32)
        + jnp.dot(x_ref[...], wx_ref[...], preferred_element_type=jnp.float32)
        + cst + b_ref[...], 0.0)
    hv_s[pl.ds(i * BN, BN), :] = hv
    part = jnp.concatenate(
        [jnp.sum(pk, axis=0, keepdims=True),
         jnp.sum(hv, axis=0, keepdims=True)], axis=0)

    @pl.when(i == 0)
    def _():
      acc_ref[...] = jnp.zeros_like(acc_ref)

    acc_ref[0:2, 0:H] += part

  @pl.when(i >= 5)
  def _():
    ii = i - 5
    pk = agg_s[pl.ds(ii * BN, BN), :]
    hv = hv_s[pl.ds(ii * BN, BN), :]
    mean_he = acc_ref[0:1, 0:H] * (1.0 / E)
    mean_hv = acc_ref[1:2, 0:H] * (1.0 / N)
    g_new = jnp.maximum(
        jnp.dot(mean_he, wg1_ref[...], preferred_element_type=jnp.float32)
        + jnp.dot(mean_hv, wg2_ref[...], preferred_element_type=jnp.float32)
        + jnp.dot(g_ref[...], wg3_ref[...], preferred_element_type=jnp.float32)
        + bgb_ref[...], 0.0)
    gterm = (jnp.dot(g_new, wgn_ref[...], preferred_element_type=jnp.float32)
             + b2a_ref[...])
    h2 = jnp.maximum(
        jnp.dot(pk, wa2_ref[...], preferred_element_type=jnp.float32)
        + jnp.dot(hv, wv2_ref[...], preferred_element_type=jnp.float32)
        + gterm, 0.0)
    out_ref[...] = (jnp.dot(h2, w2b_ref[...], preferred_element_type=jnp.float32)
                    + b2b_ref[...])

    @pl.when(i == 5)
    def _():
      gnew_ref[...] = g_new


def _full(shape):
  nd = len(shape)
  return pl.BlockSpec(shape, lambda i: (0,) * nd)


def kernel(cat_x, cat_e, edge_index, global_attr, W_eb, b_eb, W_nb, b_nb,
           W_gb, b_gb, W_n2a, b_n2a, W_n2b, b_n2b):
  IN_X = cat_x.shape[1]
  IN_E = cat_e.shape[1]
  G = global_attr.shape[0]
  senders = edge_index[0]
  receivers = edge_index[1]
  g_row = global_attr.reshape(1, G)

  W_eb_e = W_eb[:IN_E]
  W_eb_x = W_eb[IN_E:IN_E + IN_X]
  W_eb_g = W_eb[IN_E + IN_X:]

  BE = 8000
  xproj, ceproj = pl.pallas_call(
      _proj_body,
      grid=(1 + E // BE,),
      in_specs=[_full((N, IN_X)), _full((IN_X, H)),
                _full((1, G)), _full((G, H)), _full((1, H)),
                pl.BlockSpec((BE, IN_E), lambda i: (jnp.maximum(i - 1, 0), 0)),
                _full((IN_E, H))],
      out_specs=[_full((N, HP)),
                 pl.BlockSpec((BE, H), lambda i: (jnp.maximum(i - 1, 0), 0))],
      out_shape=[jax.ShapeDtypeStruct((N, HP), jnp.float32),
                 jax.ShapeDtypeStruct((E, H), jnp.float32)],
  )(cat_x, W_eb_x, g_row, W_eb_g, b_eb.reshape(1, H), cat_e, W_eb_e)

  agg_packed = _sc_edge_aggregate(xproj, ceproj, senders, receivers)

  W_nb_a = W_nb[:H]
  W_nb_x = W_nb[H:H + IN_X]
  W_nb_g = W_nb[H + IN_X:]
  OUT = W_n2b.shape[1]
  W_gb1 = W_gb[:H]
  W_gb2 = W_gb[H:2 * H]
  W_gb3 = W_gb[2 * H:]
  W_n2a_a = W_n2a[:H]
  W_n2a_v = W_n2a[H:2 * H]
  W_n2a_g = W_n2a[2 * H:]
  BN = 2000
  out_nodes, g_new = pl.pallas_call(
      _node_body,
      grid=(10,),
      in_specs=[pl.BlockSpec((NC, BN, HP), lambda i: (0, jnp.minimum(i, 4), 0)),
                pl.BlockSpec((BN, IN_X), lambda i: (jnp.minimum(i, 4), 0)),
                _full((H, H)), _full((IN_X, H)), _full((1, G)), _full((G, H)),
                _full((1, H)),
                _full((H, G)), _full((H, G)), _full((G, G)), _full((1, G)),
                _full((H, H)), _full((H, H)), _full((G, H)), _full((1, H)),
                _full((H, OUT)), _full((1, OUT))],
      out_specs=[pl.BlockSpec((BN, OUT), lambda i: (jnp.maximum(i - 5, 0), 0)),
                 _full((1, G))],
      out_shape=[jax.ShapeDtypeStruct((N, OUT), jnp.float32),
                 jax.ShapeDtypeStruct((1, G), jnp.float32)],
      scratch_shapes=[pltpu.VMEM((N, H), jnp.float32),
                      pltpu.VMEM((N, H), jnp.float32),
                      pltpu.VMEM((8, 128), jnp.float32)],
  )(agg_packed, cat_x, W_nb_a, W_nb_x, g_row, W_nb_g, b_nb.reshape(1, H),
    W_gb1, W_gb2, W_gb3, b_gb.reshape(1, G),
    W_n2a_a, W_n2a_v, W_n2a_g, b_n2a.reshape(1, H),
    W_n2b, b_n2b.reshape(1, OUT))

  return (out_nodes, g_new.reshape(G))


# speedup vs baseline: 1.5134x; 1.1042x over previous
import functools

import jax
import jax.numpy as jnp
from jax import lax
from jax.experimental import pallas as pl
from jax.experimental.pallas import tpu as pltpu
from jax.experimental.pallas import tpu_sc as plsc

N = 10000
E = 320000
H = 64
HP = 128

NC = 2
NS = 16
NW = NC * NS
EPW = E // NW
CHUNK = 80
NCHUNK = EPW // CHUNK
N_PAD = 10240
ROWS_PER_TILE = N_PAD // NS
ZROWS = 128


def _sc_edge_aggregate(xproj, ceproj, senders, receivers):
  mesh = plsc.VectorSubcoreMesh(core_axis_name="c", subcore_axis_name="s")

  @functools.partial(
      pl.kernel,
      out_type=jax.ShapeDtypeStruct((NC, N_PAD, HP), jnp.float32),
      mesh=mesh,
      scratch_types=[
          pltpu.VMEM((1, CHUNK), jnp.int32),
          pltpu.VMEM((1, CHUNK), jnp.int32),
          pltpu.VMEM((1, CHUNK), jnp.int32),
          pltpu.VMEM((1, CHUNK), jnp.int32),
          pltpu.VMEM((CHUNK, H), jnp.float32),
          pltpu.VMEM((CHUNK, HP), jnp.float32),
          pltpu.VMEM((CHUNK, HP), jnp.float32),
          pltpu.VMEM((ZROWS, HP), jnp.float32),
          pltpu.VMEM_SHARED((N_PAD, HP), jnp.float32),
          pltpu.SemaphoreType.DMA,
          pltpu.SemaphoreType.DMA,
          pltpu.SemaphoreType.DMA,
          pltpu.SemaphoreType.DMA,
          pltpu.SemaphoreType.DMA,
          pltpu.SemaphoreType.DMA,
          pltpu.SemaphoreType.DMA,
      ],
  )
  def k(xproj_hbm, ceproj_hbm, snd_hbm, rcv_hbm, out_hbm,
        sidx0, sidx1, ridx0, ridx1, ce, xs, he, zbuf, acc,
        sem_ce, sem_g, sem_s0, sem_s1, sem_r0, sem_r1, sem_sc):
    c = lax.axis_index("c")
    s = lax.axis_index("s")
    wid = s * NC + c
    ebase = wid * EPW
    sidx = (sidx0, sidx1)
    ridx = (ridx0, ridx1)
    sem_s = (sem_s0, sem_s1)
    sem_r = (sem_r0, sem_r1)

    def zrow(r, carry):
      for q in range(HP // 16):
        zbuf[r, pl.ds(q * 16, 16)] = jnp.zeros((16,), jnp.float32)
      return carry
    lax.fori_loop(0, ZROWS, zrow, 0)

    def zhe(r, carry):
      for q in range(H // 16, HP // 16):
        he[r, pl.ds(q * 16, 16)] = jnp.zeros((16,), jnp.float32)
      return carry
    lax.fori_loop(0, CHUNK, zhe, 0)
    for kk in range(ROWS_PER_TILE // ZROWS):
      pltpu.sync_copy(zbuf, acc.at[pl.ds(s * ROWS_PER_TILE + kk * ZROWS, ZROWS)])
    plsc.subcore_barrier()

    def fetch_idx(j, p):
      pltpu.async_copy(snd_hbm.at[pl.ds(ebase + j * CHUNK, CHUNK)],
                       sidx[p].at[0], sem_s[p])
      pltpu.async_copy(rcv_hbm.at[pl.ds(ebase + j * CHUNK, CHUNK)],
                       ridx[p].at[0], sem_r[p])

    def work(j, p, wait_sc):
      pltpu.make_async_copy(snd_hbm.at[pl.ds(0, CHUNK)],
                            sidx[p].at[0], sem_s[p]).wait()
      pltpu.async_copy(ceproj_hbm.at[pl.ds(ebase + j * CHUNK, CHUNK)],
                       ce, sem_ce)
      pltpu.async_copy(xproj_hbm.at[sidx[p].at[0]], xs, sem_g)
      if wait_sc:
        pltpu.make_async_copy(he, acc.at[ridx[p].at[0]], sem_sc).wait()
      pltpu.make_async_copy(ceproj_hbm.at[pl.ds(ebase + j * CHUNK, CHUNK)],
                            ce, sem_ce).wait()
      pltpu.make_async_copy(xproj_hbm.at[pl.ds(0, CHUNK)], xs, sem_g).wait()

      def row(r, rc):
        for q in range(H // 16):
          sl = pl.ds(q * 16, 16)
          he[r, sl] = jnp.maximum(ce[r, sl] + xs[r, sl], 0.0)
        return rc
      lax.fori_loop(0, CHUNK, row, 0)

      pltpu.make_async_copy(rcv_hbm.at[pl.ds(0, CHUNK)],
                            ridx[p].at[0], sem_r[p]).wait()
      pltpu.async_copy(he, acc.at[ridx[p].at[0]], sem_sc, add=True)

    fetch_idx(0, 0)
    fetch_idx(1, 1)
    work(0, 0, False)
    fetch_idx(2, 0)

    def two(t, carry):
      j = 2 * t + 1
      work(j, 1, True)
      fetch_idx(j + 2, 1)
      work(j + 1, 0, True)
      fetch_idx(j + 3, 0)
      return carry
    lax.fori_loop(0, (NCHUNK - 3) // 2, two, 0)
    work(NCHUNK - 2, 1, True)
    work(NCHUNK - 1, 0, True)
    pltpu.make_async_copy(he, acc.at[ridx[0].at[0]], sem_sc).wait()
    plsc.subcore_barrier()

    pltpu.sync_copy(acc.at[pl.ds(s * ROWS_PER_TILE, ROWS_PER_TILE)],
                    out_hbm.at[c, pl.ds(s * ROWS_PER_TILE, ROWS_PER_TILE)])

  return k(xproj, ceproj, senders, receivers)


def _proj_body(x_ref, wx_ref, g_ref, wg_ref, b_ref, e_ref, we_ref,
               xo_ref, ceo_ref):
  i = pl.program_id(0)

  @pl.when(i == 0)
  def _():
    cst = jnp.dot(g_ref[...], wg_ref[...], preferred_element_type=jnp.float32)
    proj = (jnp.dot(x_ref[...], wx_ref[...],
                    preferred_element_type=jnp.float32) + cst + b_ref[...])
    xo_ref[...] = jnp.concatenate(
        [proj, jnp.zeros((proj.shape[0], HP - H), jnp.float32)], axis=1)

  @pl.when(i > 0)
  def _():
    ceo_ref[...] = jnp.dot(e_ref[...], we_ref[...],
                           preferred_element_type=jnp.float32)


def _node_body(aggp_ref, x_ref, wa_ref, wx_ref, g_ref, wg_ref, b_ref,
               wg1_ref, wg2_ref, wg3_ref, bgb_ref,
               wa2_ref, wv2_ref, wgn_ref, b2a_ref, w2b_ref, b2b_ref,
               out_ref, gnew_ref, agg_s, hv_s, acc_ref):
  i = pl.program_id(0)
  BN = aggp_ref.shape[1]

  @pl.when(i < 5)
  def _():
    pk = aggp_ref[0, :, :H] + aggp_ref[1, :, :H]
    agg_s[pl.ds(i * BN, BN), :] = pk
    cst = jnp.dot(g_ref[...], wg_ref[...], preferred_element_type=jnp.float32)
    hv = jnp.maximum(
        jnp.dot(pk, wa_ref[...], preferred_element_type=jnp.float32)
        + jnp.dot(x_ref[...], wx_ref[...], preferred_element_type=jnp.float32)
        + cst + b_ref[...], 0.0)
    hv_s[pl.ds(i * BN, BN), :] = hv
    part = jnp.concatenate(
        [jnp.sum(pk, axis=0, keepdims=True),
         jnp.sum(hv, axis=0, keepdims=True)], axis=0)

    @pl.when(i == 0)
    def _():
      acc_ref[...] = jnp.zeros_like(acc_ref)

    acc_ref[0:2, 0:H] += part

  @pl.when(i >= 5)
  def _():
    ii = i - 5
    pk = agg_s[pl.ds(ii * BN, BN), :]
    hv = hv_s[pl.ds(ii * BN, BN), :]
    mean_he = acc_ref[0:1, 0:H] * (1.0 / E)
    mean_hv = acc_ref[1:2, 0:H] * (1.0 / N)
    g_new = jnp.maximum(
        jnp.dot(mean_he, wg1_ref[...], preferred_element_type=jnp.float32)
        + jnp.dot(mean_hv, wg2_ref[...], preferred_element_type=jnp.float32)
        + jnp.dot(g_ref[...], wg3_ref[...], preferred_element_type=jnp.float32)
        + bgb_ref[...], 0.0)
    gterm = (jnp.dot(g_new, wgn_ref[...], preferred_element_type=jnp.float32)
             + b2a_ref[...])
    h2 = jnp.maximum(
        jnp.dot(pk, wa2_ref[...], preferred_element_type=jnp.float32)
        + jnp.dot(hv, wv2_ref[...], preferred_element_type=jnp.float32)
        + gterm, 0.0)
    out_ref[...] = (jnp.dot(h2, w2b_ref[...], preferred_element_type=jnp.float32)
                    + b2b_ref[...])

    @pl.when(i == 5)
    def _():
      gnew_ref[...] = g_new


def _full(shape):
  nd = len(shape)
  return pl.BlockSpec(shape, lambda i: (0,) * nd)


def kernel(cat_x, cat_e, edge_index, global_attr, W_eb, b_eb, W_nb, b_nb,
           W_gb, b_gb, W_n2a, b_n2a, W_n2b, b_n2b):
  IN_X = cat_x.shape[1]
  IN_E = cat_e.shape[1]
  G = global_attr.shape[0]
  senders = edge_index[0]
  receivers = edge_index[1]
  g_row = global_attr.reshape(1, G)

  W_eb_e = W_eb[:IN_E]
  W_eb_x = W_eb[IN_E:IN_E + IN_X]
  W_eb_g = W_eb[IN_E + IN_X:]

  BE = 8000
  xproj, ceproj = pl.pallas_call(
      _proj_body,
      grid=(1 + E // BE,),
      in_specs=[_full((N, IN_X)), _full((IN_X, H)),
                _full((1, G)), _full((G, H)), _full((1, H)),
                pl.BlockSpec((BE, IN_E), lambda i: (jnp.maximum(i - 1, 0), 0)),
                _full((IN_E, H))],
      out_specs=[_full((N, HP)),
                 pl.BlockSpec((BE, H), lambda i: (jnp.maximum(i - 1, 0), 0))],
      out_shape=[jax.ShapeDtypeStruct((N, HP), jnp.float32),
                 jax.ShapeDtypeStruct((E, H), jnp.float32)],
  )(cat_x, W_eb_x, g_row, W_eb_g, b_eb.reshape(1, H), cat_e, W_eb_e)

  agg_packed = _sc_edge_aggregate(xproj, ceproj, senders, receivers)

  W_nb_a = W_nb[:H]
  W_nb_x = W_nb[H:H + IN_X]
  W_nb_g = W_nb[H + IN_X:]
  OUT = W_n2b.shape[1]
  W_gb1 = W_gb[:H]
  W_gb2 = W_gb[H:2 * H]
  W_gb3 = W_gb[2 * H:]
  W_n2a_a = W_n2a[:H]
  W_n2a_v = W_n2a[H:2 * H]
  W_n2a_g = W_n2a[2 * H:]
  BN = 2000
  out_nodes, g_new = pl.pallas_call(
      _node_body,
      grid=(10,),
      in_specs=[pl.BlockSpec((NC, BN, HP), lambda i: (0, jnp.minimum(i, 4), 0)),
                pl.BlockSpec((BN, IN_X), lambda i: (jnp.minimum(i, 4), 0)),
                _full((H, H)), _full((IN_X, H)), _full((1, G)), _full((G, H)),
                _full((1, H)),
                _full((H, G)), _full((H, G)), _full((G, G)), _full((1, G)),
                _full((H, H)), _full((H, H)), _full((G, H)), _full((1, H)),
                _full((H, OUT)), _full((1, OUT))],
      out_specs=[pl.BlockSpec((BN, OUT), lambda i: (jnp.maximum(i - 5, 0), 0)),
                 _full((1, G))],
      out_shape=[jax.ShapeDtypeStruct((N, OUT), jnp.float32),
                 jax.ShapeDtypeStruct((1, G), jnp.float32)],
      scratch_shapes=[pltpu.VMEM((N, H), jnp.float32),
                      pltpu.VMEM((N, H), jnp.float32),
                      pltpu.VMEM((8, 128), jnp.float32)],
  )(agg_packed, cat_x, W_nb_a, W_nb_x, g_row, W_nb_g, b_nb.reshape(1, H),
    W_gb1, W_gb2, W_gb3, b_gb.reshape(1, G),
    W_n2a_a, W_n2a_v, W_n2a_g, b_n2a.reshape(1, H),
    W_n2b, b_n2b.reshape(1, OUT))

  return (out_nodes, g_new.reshape(G))


# speedup vs baseline: 1.5184x; 1.0033x over previous
import functools

import jax
import jax.numpy as jnp
from jax import lax
from jax.experimental import pallas as pl
from jax.experimental.pallas import tpu as pltpu
from jax.experimental.pallas import tpu_sc as plsc

N = 10000
E = 320000
H = 64
HP = 128

NC = 2
NS = 16
NW = NC * NS
EPW = E // NW
CHUNK = 80
NCHUNK = EPW // CHUNK
N_PAD = 10240
ROWS_PER_TILE = N_PAD // NS
ZROWS = 128


def _sc_edge_aggregate(xproj, ceproj, senders, receivers):
  mesh = plsc.VectorSubcoreMesh(core_axis_name="c", subcore_axis_name="s")

  @functools.partial(
      pl.kernel,
      out_type=jax.ShapeDtypeStruct((NC, N_PAD, HP), jnp.float32),
      mesh=mesh,
      scratch_types=[
          pltpu.VMEM((1, CHUNK), jnp.int32),
          pltpu.VMEM((1, CHUNK), jnp.int32),
          pltpu.VMEM((1, CHUNK), jnp.int32),
          pltpu.VMEM((1, CHUNK), jnp.int32),
          pltpu.VMEM((CHUNK, H), jnp.float32),
          pltpu.VMEM((CHUNK, HP), jnp.float32),
          pltpu.VMEM((CHUNK, HP), jnp.float32),
          pltpu.VMEM((ZROWS, HP), jnp.float32),
          pltpu.VMEM_SHARED((N_PAD, HP), jnp.float32),
          pltpu.SemaphoreType.DMA,
          pltpu.SemaphoreType.DMA,
          pltpu.SemaphoreType.DMA,
          pltpu.SemaphoreType.DMA,
          pltpu.SemaphoreType.DMA,
          pltpu.SemaphoreType.DMA,
          pltpu.SemaphoreType.DMA,
      ],
  )
  def k(xproj_hbm, ceproj_hbm, snd_hbm, rcv_hbm, out_hbm,
        sidx0, sidx1, ridx0, ridx1, ce, xs, he, zbuf, acc,
        sem_ce, sem_g, sem_s0, sem_s1, sem_r0, sem_r1, sem_sc):
    c = lax.axis_index("c")
    s = lax.axis_index("s")
    wid = s * NC + c
    ebase = wid * EPW
    sidx = (sidx0, sidx1)
    ridx = (ridx0, ridx1)
    sem_s = (sem_s0, sem_s1)
    sem_r = (sem_r0, sem_r1)

    def zrow(r, carry):
      for q in range(HP // 16):
        zbuf[r, pl.ds(q * 16, 16)] = jnp.zeros((16,), jnp.float32)
      return carry
    lax.fori_loop(0, ZROWS, zrow, 0)

    def zhe(r, carry):
      for q in range(H // 16, HP // 16):
        he[r, pl.ds(q * 16, 16)] = jnp.zeros((16,), jnp.float32)
      return carry
    lax.fori_loop(0, CHUNK, zhe, 0)
    for kk in range(ROWS_PER_TILE // ZROWS):
      pltpu.sync_copy(zbuf, acc.at[pl.ds(s * ROWS_PER_TILE + kk * ZROWS, ZROWS)])
    plsc.subcore_barrier()

    def fetch_idx(j, p):
      pltpu.async_copy(snd_hbm.at[pl.ds(ebase + j * CHUNK, CHUNK)],
                       sidx[p].at[0], sem_s[p])
      pltpu.async_copy(rcv_hbm.at[pl.ds(ebase + j * CHUNK, CHUNK)],
                       ridx[p].at[0], sem_r[p])

    def work(j, p, wait_sc):
      pltpu.make_async_copy(snd_hbm.at[pl.ds(0, CHUNK)],
                            sidx[p].at[0], sem_s[p]).wait()
      pltpu.async_copy(ceproj_hbm.at[pl.ds(ebase + j * CHUNK, CHUNK)],
                       ce, sem_ce)
      pltpu.async_copy(xproj_hbm.at[sidx[p].at[0]], xs, sem_g)
      if wait_sc:
        pltpu.make_async_copy(he, acc.at[ridx[p].at[0]], sem_sc).wait()
      pltpu.make_async_copy(ceproj_hbm.at[pl.ds(ebase + j * CHUNK, CHUNK)],
                            ce, sem_ce).wait()
      pltpu.make_async_copy(xproj_hbm.at[pl.ds(0, CHUNK)], xs, sem_g).wait()

      @plsc.parallel_loop(0, CHUNK, unroll=4)
      def _(r):
        for q in range(H // 16):
          sl = pl.ds(q * 16, 16)
          he[r, sl] = jnp.maximum(ce[r, sl] + xs[r, sl], 0.0)

      pltpu.make_async_copy(rcv_hbm.at[pl.ds(0, CHUNK)],
                            ridx[p].at[0], sem_r[p]).wait()
      pltpu.async_copy(he, acc.at[ridx[p].at[0]], sem_sc, add=True)

    fetch_idx(0, 0)
    fetch_idx(1, 1)
    work(0, 0, False)
    fetch_idx(2, 0)

    def two(t, carry):
      j = 2 * t + 1
      work(j, 1, True)
      fetch_idx(j + 2, 1)
      work(j + 1, 0, True)
      fetch_idx(j + 3, 0)
      return carry
    lax.fori_loop(0, (NCHUNK - 3) // 2, two, 0)
    work(NCHUNK - 2, 1, True)
    work(NCHUNK - 1, 0, True)
    pltpu.make_async_copy(he, acc.at[ridx[0].at[0]], sem_sc).wait()
    plsc.subcore_barrier()

    pltpu.sync_copy(acc.at[pl.ds(s * ROWS_PER_TILE, ROWS_PER_TILE)],
                    out_hbm.at[c, pl.ds(s * ROWS_PER_TILE, ROWS_PER_TILE)])

  return k(xproj, ceproj, senders, receivers)


def _proj_body(x_ref, wx_ref, g_ref, wg_ref, b_ref, e_ref, we_ref,
               xo_ref, ceo_ref):
  i = pl.program_id(0)

  @pl.when(i == 0)
  def _():
    cst = jnp.dot(g_ref[...], wg_ref[...], preferred_element_type=jnp.float32)
    proj = (jnp.dot(x_ref[...], wx_ref[...],
                    preferred_element_type=jnp.float32) + cst + b_ref[...])
    xo_ref[...] = jnp.concatenate(
        [proj, jnp.zeros((proj.shape[0], HP - H), jnp.float32)], axis=1)

  @pl.when(i > 0)
  def _():
    ceo_ref[...] = jnp.dot(e_ref[...], we_ref[...],
                           preferred_element_type=jnp.float32)


def _node_body(aggp_ref, x_ref, wa_ref, wx_ref, g_ref, wg_ref, b_ref,
               wg1_ref, wg2_ref, wg3_ref, bgb_ref,
               wa2_ref, wv2_ref, wgn_ref, b2a_ref, w2b_ref, b2b_ref,
               out_ref, gnew_ref, agg_s, hv_s, acc_ref):
  i = pl.program_id(0)
  BN = aggp_ref.shape[1]

  @pl.when(i < 5)
  def _():
    pk = aggp_ref[0, :, :H] + aggp_ref[1, :, :H]
    agg_s[pl.ds(i * BN, BN), :] = pk
    cst = jnp.dot(g_ref[...], wg_ref[...], preferred_element_type=jnp.float32)
    hv = jnp.maximum(
        jnp.dot(pk, wa_ref[...], preferred_element_type=jnp.float32)
        + jnp.dot(x_ref[...], wx_ref[...], preferred_element_type=jnp.float32)
        + cst + b_ref[...], 0.0)
    hv_s[pl.ds(i * BN, BN), :] = hv
    part = jnp.concatenate(
        [jnp.sum(pk, axis=0, keepdims=True),
         jnp.sum(hv, axis=0, keepdims=True)], axis=0)

    @pl.when(i == 0)
    def _():
      acc_ref[...] = jnp.zeros_like(acc_ref)

    acc_ref[0:2, 0:H] += part

  @pl.when(i >= 5)
  def _():
    ii = i - 5
    pk = agg_s[pl.ds(ii * BN, BN), :]
    hv = hv_s[pl.ds(ii * BN, BN), :]
    mean_he = acc_ref[0:1, 0:H] * (1.0 / E)
    mean_hv = acc_ref[1:2, 0:H] * (1.0 / N)
    g_new = jnp.maximum(
        jnp.dot(mean_he, wg1_ref[...], preferred_element_type=jnp.float32)
        + jnp.dot(mean_hv, wg2_ref[...], preferred_element_type=jnp.float32)
        + jnp.dot(g_ref[...], wg3_ref[...], preferred_element_type=jnp.float32)
        + bgb_ref[...], 0.0)
    gterm = (jnp.dot(g_new, wgn_ref[...], preferred_element_type=jnp.float32)
             + b2a_ref[...])
    h2 = jnp.maximum(
        jnp.dot(pk, wa2_ref[...], preferred_element_type=jnp.float32)
        + jnp.dot(hv, wv2_ref[...], preferred_element_type=jnp.float32)
        + gterm, 0.0)
    out_ref[...] = (jnp.dot(h2, w2b_ref[...], preferred_element_type=jnp.float32)
                    + b2b_ref[...])

    @pl.when(i == 5)
    def _():
      gnew_ref[...] = g_new


def _full(shape):
  nd = len(shape)
  return pl.BlockSpec(shape, lambda i: (0,) * nd)


def kernel(cat_x, cat_e, edge_index, global_attr, W_eb, b_eb, W_nb, b_nb,
           W_gb, b_gb, W_n2a, b_n2a, W_n2b, b_n2b):
  IN_X = cat_x.shape[1]
  IN_E = cat_e.shape[1]
  G = global_attr.shape[0]
  senders = edge_index[0]
  receivers = edge_index[1]
  g_row = global_attr.reshape(1, G)

  W_eb_e = W_eb[:IN_E]
  W_eb_x = W_eb[IN_E:IN_E + IN_X]
  W_eb_g = W_eb[IN_E + IN_X:]

  BE = 16000
  xproj, ceproj = pl.pallas_call(
      _proj_body,
      grid=(1 + E // BE,),
      in_specs=[_full((N, IN_X)), _full((IN_X, H)),
                _full((1, G)), _full((G, H)), _full((1, H)),
                pl.BlockSpec((BE, IN_E), lambda i: (jnp.maximum(i - 1, 0), 0)),
                _full((IN_E, H))],
      out_specs=[_full((N, HP)),
                 pl.BlockSpec((BE, H), lambda i: (jnp.maximum(i - 1, 0), 0))],
      out_shape=[jax.ShapeDtypeStruct((N, HP), jnp.float32),
                 jax.ShapeDtypeStruct((E, H), jnp.float32)],
  )(cat_x, W_eb_x, g_row, W_eb_g, b_eb.reshape(1, H), cat_e, W_eb_e)

  agg_packed = _sc_edge_aggregate(xproj, ceproj, senders, receivers)

  W_nb_a = W_nb[:H]
  W_nb_x = W_nb[H:H + IN_X]
  W_nb_g = W_nb[H + IN_X:]
  OUT = W_n2b.shape[1]
  W_gb1 = W_gb[:H]
  W_gb2 = W_gb[H:2 * H]
  W_gb3 = W_gb[2 * H:]
  W_n2a_a = W_n2a[:H]
  W_n2a_v = W_n2a[H:2 * H]
  W_n2a_g = W_n2a[2 * H:]
  BN = 2000
  out_nodes, g_new = pl.pallas_call(
      _node_body,
      grid=(10,),
      in_specs=[pl.BlockSpec((NC, BN, HP), lambda i: (0, jnp.minimum(i, 4), 0)),
                pl.BlockSpec((BN, IN_X), lambda i: (jnp.minimum(i, 4), 0)),
                _full((H, H)), _full((IN_X, H)), _full((1, G)), _full((G, H)),
                _full((1, H)),
                _full((H, G)), _full((H, G)), _full((G, G)), _full((1, G)),
                _full((H, H)), _full((H, H)), _full((G, H)), _full((1, H)),
                _full((H, OUT)), _full((1, OUT))],
      out_specs=[pl.BlockSpec((BN, OUT), lambda i: (jnp.maximum(i - 5, 0), 0)),
                 _full((1, G))],
      out_shape=[jax.ShapeDtypeStruct((N, OUT), jnp.float32),
                 jax.ShapeDtypeStruct((1, G), jnp.float32)],
      scratch_shapes=[pltpu.VMEM((N, H), jnp.float32),
                      pltpu.VMEM((N, H), jnp.float32),
                      pltpu.VMEM((8, 128), jnp.float32)],
  )(agg_packed, cat_x, W_nb_a, W_nb_x, g_row, W_nb_g, b_nb.reshape(1, H),
    W_gb1, W_gb2, W_gb3, b_gb.reshape(1, G),
    W_n2a_a, W_n2a_v, W_n2a_g, b_n2a.reshape(1, H),
    W_n2b, b_n2b.reshape(1, OUT))

  return (out_nodes, g_new.reshape(G))
